# baseline (speedup 1.0000x reference)
import jax
import jax.numpy as jnp
from jax import lax
from jax.experimental import pallas as pl
from jax.experimental.pallas import tpu as pltpu
from jax.experimental.pallas import tpu_sc as plsc
import functools

L = 16
NC, NS = 2, 16
NW = NC * NS
NBIS = 30
G = 32
CAP = 1024


def _tree_max(vs):
    vs = list(vs)
    while len(vs) > 1:
        nxt = [jnp.maximum(vs[i], vs[i + 1]) for i in range(0, len(vs) - 1, 2)]
        if len(vs) % 2:
            nxt.append(vs[-1])
        vs = nxt
    return vs[0]


def _sparsegen_rows(nrows, dim):
    rpw = nrows // NW
    nchunk = dim // L
    ngrp = nchunk // G
    mesh = plsc.VectorSubcoreMesh(core_axis_name="c", subcore_axis_name="s")

    @functools.partial(
        pl.kernel,
        out_type=jax.ShapeDtypeStruct((nrows, dim), jnp.float32),
        mesh=mesh,
        compiler_params=pltpu.CompilerParams(needs_layout_passes=False),
        scratch_types=[
            pltpu.VMEM((dim,), jnp.float32),
            pltpu.VMEM((dim,), jnp.float32),
            pltpu.VMEM((dim,), jnp.float32),
            pltpu.VMEM((ngrp * L,), jnp.float32),
            pltpu.VMEM(((ngrp // G) * L,), jnp.float32),
            pltpu.VMEM((CAP + L,), jnp.float32),
            pltpu.VMEM((CAP + L,), jnp.int32),
            pltpu.VMEM((CAP + L,), jnp.int32),
            pltpu.SemaphoreType.DMA,
            pltpu.SemaphoreType.DMA,
            pltpu.SemaphoreType.DMA,
        ],
    )
    def k(x_hbm, out_hbm, xz0, xz1, outb, gm, gm2, candz, candi0, candi1,
          in_sem0, in_sem1, out_sem):
        wid = lax.axis_index("s") * NC + lax.axis_index("c")
        zeros = jnp.zeros((L,), jnp.float32)
        iota = lax.iota(jnp.int32, L)
        xzs = [xz0, xz1]
        candis = [candi0, candi1]
        in_sems = [in_sem0, in_sem1]

        def in_copy(r, buf):
            return pltpu.make_async_copy(x_hbm.at[wid * rpw + r], buf,
                                         in_sems[r % 2])

        def out_copy(r):
            return pltpu.make_async_copy(outb, out_hbm.at[wid * rpw + r],
                                         out_sem)

        in_copy(0, xz0).start()

        @plsc.parallel_loop(0, nchunk, unroll=8)
        def _zout(i):
            outb[pl.ds(i * L, L)] = zeros

        def bisect(buf, ncc, lo0, hi0):
            def bis(t, c):
                lo, hi = c
                mid = 0.5 * (lo + hi)

                def sw(i, acc):
                    z = buf[pl.ds(i * L, L)]
                    return acc + jnp.maximum(z - mid, 0.0)

                f = jnp.sum(lax.fori_loop(0, ncc, sw, zeros)) - 1.0
                pred = f >= 0.0
                return (jnp.where(pred, mid, lo), jnp.where(pred, hi, mid))

            lo, _hi = lax.fori_loop(0, NBIS, bis, (lo0, hi0))

            def rsw(i, c):
                s, kk = c
                z = buf[pl.ds(i * L, L)]
                m2 = z > lo
                return (s + jnp.where(m2, z, 0.0),
                        kk + jnp.where(m2, 1.0, 0.0))

            s, kk = lax.fori_loop(0, ncc, rsw, (zeros, zeros))
            return ((jnp.full((L,), jnp.sum(s), jnp.float32) - 1.0)
                    / jnp.full((L,), jnp.sum(kk), jnp.float32))

        prev_normal = None

        for r in range(rpw):
            xz = xzs[r % 2]
            candi = candis[r % 2]
            if r + 1 < rpw:
                in_copy(r + 1, xzs[(r + 1) % 2]).start()
            in_copy(r, xz).wait()

            @plsc.parallel_loop(0, ngrp, unroll=1, carry=jnp.full(
                (L,), -3.0e38, jnp.float32))
            def m(g, mc):
                base = g * (G * L)
                xs = [xz[pl.ds(base + c * L, L)] for c in range(G)]
                gmax = _tree_max(xs)
                gm[pl.ds(g * L, L)] = gmax
                return jnp.maximum(mc, gmax)

            xmax = jnp.max(m)
            zv = jnp.exp(jnp.full((L,), xmax, jnp.float32))
            zmax = zv[0]
            lo0 = zmax - 1.0
            uv = 1.0 / (zv - 1.0)
            xthr = jnp.where(zmax > 1.0,
                             xmax - uv[0] - (jnp.abs(xmax) + 1.0) * 1e-6,
                             -3.0e38)

            @plsc.parallel_loop(0, ngrp // G)
            def _gm2(j):
                vs = [gm[pl.ds((j * G + t) * L, L)] for t in range(G)]
                gm2[pl.ds(j * L, L)] = _tree_max(vs)

            if prev_normal is not None:
                p_norm, p_ncand, p_candi, p_r = prev_normal
                out_copy(p_r).wait()

                @pl.when(p_norm)
                def _():
                    p_ncc = lax.shift_right_logical(p_ncand + (L - 1), 4)

                    def zf(i, _):
                        base = i * L
                        mv = (base + iota) < p_ncand
                        plsc.store_scatter(
                            outb, [p_candi[pl.ds(base, L)]], zeros, mask=mv)
                        return 0

                    lax.fori_loop(0, p_ncc, zf, 0)

                @pl.when(jnp.logical_not(p_norm))
                def _():
                    @plsc.parallel_loop(0, nchunk, unroll=8)
                    def _rz(i):
                        outb[pl.ds(i * L, L)] = zeros

            def b2_grp(g, off):
                hot = jnp.any(gm[pl.ds(g * L, L)] > xthr)

                def hot_fn(off):
                    def chunk_fn(c, off):
                        ci = g * G + c
                        xv = xz[pl.ds(ci * L, L)]
                        mk = xv > xthr

                        @pl.when(off <= CAP)
                        def _():
                            plsc.store_compressed(
                                candz.at[pl.ds(off, L)], jnp.exp(xv), mask=mk)
                            plsc.store_compressed(
                                candi.at[pl.ds(off, L)], ci * L + iota,
                                mask=mk)

                        cnt = plsc.all_reduce_population_count(mk)
                        return off + cnt[0]

                    return lax.fori_loop(0, G, chunk_fn, off)

                return lax.cond(hot, hot_fn, lambda o: o, off)

            def b2_super(j, off):
                hot2 = jnp.any(gm2[pl.ds(j * L, L)] > xthr)
                return lax.cond(
                    hot2,
                    lambda o: lax.fori_loop(
                        0, G, lambda t, oo: b2_grp(j * G + t, oo), o),
                    lambda o: o, off)

            ncand = lax.fori_loop(0, ngrp // G, b2_super, jnp.int32(0))
            normal = ncand <= CAP

            @pl.when(normal)
            def _():
                candz[pl.ds(ncand, L)] = jnp.full((L,), lo0, jnp.float32)

            ncc = lax.shift_right_logical(ncand + (L - 1), 4)

            def sort_solve():
                zv = candz[pl.ds(0, L)]
                zv = jnp.where(iota < ncand, zv, -3.0e38)
                zs, _ = plsc.sort_key_val(zv, zv, descending=True)
                cs = plsc.cumsum(zs)
                kv = (iota + 1).astype(jnp.float32)
                chk = (1.0 + kv * zs) > cs
                kz = plsc.all_reduce_population_count(chk).astype(jnp.float32)
                tausum = jnp.sum(jnp.where(chk, zs, 0.0))
                return (jnp.full((L,), tausum, jnp.float32) - 1.0) / kz

            def fallback_solve():
                @plsc.parallel_loop(0, nchunk, unroll=4)
                def _ex(i):
                    xz[pl.ds(i * L, L)] = jnp.exp(xz[pl.ds(i * L, L)])

                return bisect(xz, nchunk, lo0, zmax)

            tau = lax.cond(
                ncand <= L,
                sort_solve,
                lambda: lax.cond(
                    normal,
                    lambda: bisect(candz, ncc, lo0, zmax),
                    fallback_solve))

            def c_normal():
                def cf(i, _):
                    base = i * L
                    z = candz[pl.ds(base, L)]
                    pc = jnp.maximum(z - tau, 0.0)
                    mv = (base + iota) < ncand
                    plsc.store_scatter(
                        outb, [candi[pl.ds(base, L)]], pc, mask=mv)
                    return 0

                lax.fori_loop(0, ncc, cf, 0)

            def c_fallback():
                @plsc.parallel_loop(0, nchunk, unroll=8)
                def _cd(i):
                    z = xz[pl.ds(i * L, L)]
                    outb[pl.ds(i * L, L)] = jnp.maximum(z - tau, 0.0)

            lax.cond(normal, c_normal, c_fallback)
            out_copy(r).start()
            prev_normal = (normal, ncand, candi, r)

        out_copy(rpw - 1).wait()

    return k


def kernel(input):
    nrows, dim = input.shape
    return _sparsegen_rows(nrows, dim)(input)


# speedup vs baseline: 1.0271x; 1.0271x over previous
import jax
import jax.numpy as jnp
from jax import lax
from jax.experimental import pallas as pl
from jax.experimental.pallas import tpu as pltpu
from jax.experimental.pallas import tpu_sc as plsc
import functools

L = 16
NC, NS = 2, 16
NW = NC * NS
NBIS = 30
G = 16
CAP = 1024


def _tree_max(vs):
    vs = list(vs)
    while len(vs) > 1:
        nxt = [jnp.maximum(vs[i], vs[i + 1]) for i in range(0, len(vs) - 1, 2)]
        if len(vs) % 2:
            nxt.append(vs[-1])
        vs = nxt
    return vs[0]


def _sparsegen_rows(nrows, dim):
    rpw = nrows // NW
    nchunk = dim // L
    ngrp = nchunk // G
    mesh = plsc.VectorSubcoreMesh(core_axis_name="c", subcore_axis_name="s")

    @functools.partial(
        pl.kernel,
        out_type=jax.ShapeDtypeStruct((nrows, dim), jnp.float32),
        mesh=mesh,
        compiler_params=pltpu.CompilerParams(needs_layout_passes=False),
        scratch_types=[
            pltpu.VMEM((dim,), jnp.float32),
            pltpu.VMEM((dim,), jnp.float32),
            pltpu.VMEM((dim,), jnp.float32),
            pltpu.VMEM((ngrp * L,), jnp.float32),
            pltpu.VMEM(((ngrp // G) * L,), jnp.float32),
            pltpu.VMEM((CAP + L,), jnp.float32),
            pltpu.VMEM((CAP + L,), jnp.int32),
            pltpu.VMEM((CAP + L,), jnp.int32),
            pltpu.SemaphoreType.DMA,
            pltpu.SemaphoreType.DMA,
            pltpu.SemaphoreType.DMA,
        ],
    )
    def k(x_hbm, out_hbm, xz0, xz1, outb, gm, gm2, candz, candi0, candi1,
          in_sem0, in_sem1, out_sem):
        wid = lax.axis_index("s") * NC + lax.axis_index("c")
        zeros = jnp.zeros((L,), jnp.float32)
        iota = lax.iota(jnp.int32, L)
        xzs = [xz0, xz1]
        candis = [candi0, candi1]
        in_sems = [in_sem0, in_sem1]

        def in_copy(r, buf):
            return pltpu.make_async_copy(x_hbm.at[wid * rpw + r], buf,
                                         in_sems[r % 2])

        def out_copy(r):
            return pltpu.make_async_copy(outb, out_hbm.at[wid * rpw + r],
                                         out_sem)

        in_copy(0, xz0).start()

        @plsc.parallel_loop(0, nchunk, unroll=8)
        def _zout(i):
            outb[pl.ds(i * L, L)] = zeros

        def bisect(buf, ncc, lo0, hi0):
            def bis(t, c):
                lo, hi = c
                mid = 0.5 * (lo + hi)

                def sw(i, acc):
                    z = buf[pl.ds(i * L, L)]
                    return acc + jnp.maximum(z - mid, 0.0)

                f = jnp.sum(lax.fori_loop(0, ncc, sw, zeros)) - 1.0
                pred = f >= 0.0
                return (jnp.where(pred, mid, lo), jnp.where(pred, hi, mid))

            lo, _hi = lax.fori_loop(0, NBIS, bis, (lo0, hi0))

            def rsw(i, c):
                s, kk = c
                z = buf[pl.ds(i * L, L)]
                m2 = z > lo
                return (s + jnp.where(m2, z, 0.0),
                        kk + jnp.where(m2, 1.0, 0.0))

            s, kk = lax.fori_loop(0, ncc, rsw, (zeros, zeros))
            return ((jnp.full((L,), jnp.sum(s), jnp.float32) - 1.0)
                    / jnp.full((L,), jnp.sum(kk), jnp.float32))

        prev_normal = None

        for r in range(rpw):
            xz = xzs[r % 2]
            candi = candis[r % 2]
            if 0 < r < rpw - 1:
                in_copy(r + 1, xzs[(r + 1) % 2]).start()
            in_copy(r, xz).wait()

            @plsc.parallel_loop(0, ngrp, unroll=2, carry=jnp.full(
                (L,), -3.0e38, jnp.float32))
            def m(g, mc):
                base = g * (G * L)
                xs = [xz[pl.ds(base + c * L, L)] for c in range(G)]
                gmax = _tree_max(xs)
                gm[pl.ds(g * L, L)] = gmax
                return jnp.maximum(mc, gmax)

            if r == 0 and rpw > 1:
                in_copy(1, xzs[1]).start()

            xmax = jnp.max(m)
            zv = jnp.exp(jnp.full((L,), xmax, jnp.float32))
            zmax = zv[0]
            lo0 = zmax - 1.0
            uv = 1.0 / (zv - 1.0)
            xthr = jnp.where(zmax > 1.0,
                             xmax - uv[0] - (jnp.abs(xmax) + 1.0) * 1e-6,
                             -3.0e38)

            @plsc.parallel_loop(0, ngrp // G)
            def _gm2(j):
                vs = [gm[pl.ds((j * G + t) * L, L)] for t in range(G)]
                gm2[pl.ds(j * L, L)] = _tree_max(vs)

            if prev_normal is not None:
                p_norm, p_ncand, p_candi, p_r = prev_normal
                out_copy(p_r).wait()

                @pl.when(p_norm)
                def _():
                    p_ncc = lax.shift_right_logical(p_ncand + (L - 1), 4)

                    def zf(i, _):
                        base = i * L
                        mv = (base + iota) < p_ncand
                        plsc.store_scatter(
                            outb, [p_candi[pl.ds(base, L)]], zeros, mask=mv)
                        return 0

                    lax.fori_loop(0, p_ncc, zf, 0)

                @pl.when(jnp.logical_not(p_norm))
                def _():
                    @plsc.parallel_loop(0, nchunk, unroll=8)
                    def _rz(i):
                        outb[pl.ds(i * L, L)] = zeros

            def b2_grp(g, off):
                hot = jnp.any(gm[pl.ds(g * L, L)] > xthr)

                def hot_fn(off):
                    def chunk_fn(c, off):
                        ci = g * G + c
                        xv = xz[pl.ds(ci * L, L)]
                        mk = xv > xthr

                        @pl.when(off <= CAP)
                        def _():
                            plsc.store_compressed(
                                candz.at[pl.ds(off, L)], jnp.exp(xv), mask=mk)
                            plsc.store_compressed(
                                candi.at[pl.ds(off, L)], ci * L + iota,
                                mask=mk)

                        cnt = plsc.all_reduce_population_count(mk)
                        return off + cnt[0]

                    return lax.fori_loop(0, G, chunk_fn, off)

                return lax.cond(hot, hot_fn, lambda o: o, off)

            def b2_super(j, off):
                hot2 = jnp.any(gm2[pl.ds(j * L, L)] > xthr)
                return lax.cond(
                    hot2,
                    lambda o: lax.fori_loop(
                        0, G, lambda t, oo: b2_grp(j * G + t, oo), o),
                    lambda o: o, off)

            ncand = lax.fori_loop(0, ngrp // G, b2_super, jnp.int32(0))
            normal = ncand <= CAP

            @pl.when(normal)
            def _():
                candz[pl.ds(ncand, L)] = jnp.full((L,), lo0, jnp.float32)

            ncc = lax.shift_right_logical(ncand + (L - 1), 4)

            def sort_solve():
                zv = candz[pl.ds(0, L)]
                zv = jnp.where(iota < ncand, zv, -3.0e38)
                zs, _ = plsc.sort_key_val(zv, zv, descending=True)
                cs = plsc.cumsum(zs)
                kv = (iota + 1).astype(jnp.float32)
                chk = (1.0 + kv * zs) > cs
                kz = plsc.all_reduce_population_count(chk).astype(jnp.float32)
                tausum = jnp.sum(jnp.where(chk, zs, 0.0))
                return (jnp.full((L,), tausum, jnp.float32) - 1.0) / kz

            def fallback_solve():
                @plsc.parallel_loop(0, nchunk, unroll=4)
                def _ex(i):
                    xz[pl.ds(i * L, L)] = jnp.exp(xz[pl.ds(i * L, L)])

                return bisect(xz, nchunk, lo0, zmax)

            tau = lax.cond(
                ncand <= L,
                sort_solve,
                lambda: lax.cond(
                    normal,
                    lambda: bisect(candz, ncc, lo0, zmax),
                    fallback_solve))

            def c_normal():
                def cf(i, _):
                    base = i * L
                    z = candz[pl.ds(base, L)]
                    pc = jnp.maximum(z - tau, 0.0)
                    mv = (base + iota) < ncand
                    plsc.store_scatter(
                        outb, [candi[pl.ds(base, L)]], pc, mask=mv)
                    return 0

                lax.fori_loop(0, ncc, cf, 0)

            def c_fallback():
                @plsc.parallel_loop(0, nchunk, unroll=8)
                def _cd(i):
                    z = xz[pl.ds(i * L, L)]
                    outb[pl.ds(i * L, L)] = jnp.maximum(z - tau, 0.0)

            lax.cond(normal, c_normal, c_fallback)
            out_copy(r).start()
            prev_normal = (normal, ncand, candi, r)

        out_copy(rpw - 1).wait()

    return k


def kernel(input):
    nrows, dim = input.shape
    return _sparsegen_rows(nrows, dim)(input)


# speedup vs baseline: 1.0615x; 1.0335x over previous
import jax
import jax.numpy as jnp
from jax import lax
from jax.experimental import pallas as pl
from jax.experimental.pallas import tpu as pltpu
from jax.experimental.pallas import tpu_sc as plsc
import functools

L = 16
NC, NS = 2, 16
NW = NC * NS
NBIS = 30
G = 16
CAP = 1024


def _tree_max(vs):
    vs = list(vs)
    while len(vs) > 1:
        nxt = [jnp.maximum(vs[i], vs[i + 1]) for i in range(0, len(vs) - 1, 2)]
        if len(vs) % 2:
            nxt.append(vs[-1])
        vs = nxt
    return vs[0]


def _sparsegen_rows(nrows, dim):
    rpw = nrows // NW
    nchunk = dim // L
    ngrp = nchunk // G
    mesh = plsc.VectorSubcoreMesh(core_axis_name="c", subcore_axis_name="s")

    @functools.partial(
        pl.kernel,
        out_type=jax.ShapeDtypeStruct((nrows, dim), jnp.float32),
        mesh=mesh,
        compiler_params=pltpu.CompilerParams(needs_layout_passes=False),
        scratch_types=[
            pltpu.VMEM((dim,), jnp.float32),
            pltpu.VMEM((dim,), jnp.float32),
            pltpu.VMEM((dim,), jnp.float32),
            pltpu.VMEM((ngrp * L,), jnp.float32),
            pltpu.VMEM(((ngrp // G) * L,), jnp.float32),
            pltpu.VMEM((CAP + L,), jnp.float32),
            pltpu.VMEM((CAP + L,), jnp.int32),
            pltpu.VMEM((CAP + L,), jnp.int32),
            pltpu.SemaphoreType.DMA,
            pltpu.SemaphoreType.DMA,
            pltpu.SemaphoreType.DMA,
        ],
    )
    def k(x_hbm, out_hbm, xz0, xz1, outb, gm, gm2, candz, candi0, candi1,
          in_sem0, in_sem1, out_sem):
        wid = lax.axis_index("s") * NC + lax.axis_index("c")
        zeros = jnp.zeros((L,), jnp.float32)
        iota = lax.iota(jnp.int32, L)
        xzs = [xz0, xz1]
        candis = [candi0, candi1]
        in_sems = [in_sem0, in_sem1]

        def in_copy(r, buf):
            return pltpu.make_async_copy(x_hbm.at[wid * rpw + r], buf,
                                         in_sems[r % 2])

        def out_copy(r):
            return pltpu.make_async_copy(outb, out_hbm.at[wid * rpw + r],
                                         out_sem)

        in_copy(0, xz0).start()

        @plsc.parallel_loop(0, nchunk, unroll=8)
        def _zout(i):
            outb[pl.ds(i * L, L)] = zeros

        def bisect(buf, ncc, lo0, hi0):
            def bis(t, c):
                lo, hi = c
                mid = 0.5 * (lo + hi)

                def sw(i, acc):
                    z = buf[pl.ds(i * L, L)]
                    return acc + jnp.maximum(z - mid, 0.0)

                f = jnp.sum(lax.fori_loop(0, ncc, sw, zeros)) - 1.0
                pred = f >= 0.0
                return (jnp.where(pred, mid, lo), jnp.where(pred, hi, mid))

            lo, _hi = lax.fori_loop(0, NBIS, bis, (lo0, hi0))

            def rsw(i, c):
                s, kk = c
                z = buf[pl.ds(i * L, L)]
                m2 = z > lo
                return (s + jnp.where(m2, z, 0.0),
                        kk + jnp.where(m2, 1.0, 0.0))

            s, kk = lax.fori_loop(0, ncc, rsw, (zeros, zeros))
            return ((jnp.full((L,), jnp.sum(s), jnp.float32) - 1.0)
                    / jnp.full((L,), jnp.sum(kk), jnp.float32))

        prev_normal = None

        for r in range(rpw):
            xz = xzs[r % 2]
            candi = candis[r % 2]
            if r + 1 < rpw:
                in_copy(r + 1, xzs[(r + 1) % 2]).start()
            in_copy(r, xz).wait()

            @plsc.parallel_loop(0, ngrp, unroll=2, carry=jnp.full(
                (L,), -3.0e38, jnp.float32))
            def m(g, mc):
                base = g * (G * L)
                xs = [xz[pl.ds(base + c * L, L)] for c in range(G)]
                gmax = _tree_max(xs)
                gm[pl.ds(g * L, L)] = gmax
                return jnp.maximum(mc, gmax)

            xmax = jnp.max(m)
            zv = jnp.exp(jnp.full((L,), xmax, jnp.float32))
            zmax = zv[0]
            lo0 = zmax - 1.0
            uv = 1.0 / (zv - 1.0)
            xthr = jnp.where(zmax > 1.0,
                             xmax - uv[0] - (jnp.abs(xmax) + 1.0) * 1e-6,
                             -3.0e38)

            @plsc.parallel_loop(0, ngrp // G)
            def _gm2(j):
                vs = [gm[pl.ds((j * G + t) * L, L)] for t in range(G)]
                gm2[pl.ds(j * L, L)] = _tree_max(vs)

            if prev_normal is not None:
                p_norm, p_ncand, p_candi, p_r = prev_normal
                out_copy(p_r).wait()

                @pl.when(p_norm)
                def _():
                    p_ncc = lax.shift_right_logical(p_ncand + (L - 1), 4)

                    def zf(i, _):
                        base = i * L
                        mv = (base + iota) < p_ncand
                        plsc.store_scatter(
                            outb, [p_candi[pl.ds(base, L)]], zeros, mask=mv)
                        return 0

                    lax.fori_loop(0, p_ncc, zf, 0)

                @pl.when(jnp.logical_not(p_norm))
                def _():
                    @plsc.parallel_loop(0, nchunk, unroll=8)
                    def _rz(i):
                        outb[pl.ds(i * L, L)] = zeros

            def b2_grp(g, off):
                hot = jnp.any(gm[pl.ds(g * L, L)] > xthr)

                def hot_fn(off):
                    def chunk_fn(c, off):
                        ci = g * G + c
                        xv = xz[pl.ds(ci * L, L)]
                        mk = xv > xthr

                        @pl.when(off <= CAP)
                        def _():
                            plsc.store_compressed(
                                candz.at[pl.ds(off, L)], jnp.exp(xv), mask=mk)
                            plsc.store_compressed(
                                candi.at[pl.ds(off, L)], ci * L + iota,
                                mask=mk)

                        cnt = plsc.all_reduce_population_count(mk)
                        return off + cnt[0]

                    return lax.fori_loop(0, G, chunk_fn, off)

                return lax.cond(hot, hot_fn, lambda o: o, off)

            def b2_super(j, off):
                hot2 = jnp.any(gm2[pl.ds(j * L, L)] > xthr)
                return lax.cond(
                    hot2,
                    lambda o: lax.fori_loop(
                        0, G, lambda t, oo: b2_grp(j * G + t, oo), o),
                    lambda o: o, off)

            ncand = lax.fori_loop(0, ngrp // G, b2_super, jnp.int32(0))
            normal = ncand <= CAP

            @pl.when(normal)
            def _():
                candz[pl.ds(ncand, L)] = jnp.full((L,), lo0, jnp.float32)

            ncc = lax.shift_right_logical(ncand + (L - 1), 4)

            def sort_solve():
                zv = candz[pl.ds(0, L)]
                zv = jnp.where(iota < ncand, zv, -3.0e38)
                zs, _ = plsc.sort_key_val(zv, zv, descending=True)
                cs = plsc.cumsum(zs)
                kv = (iota + 1).astype(jnp.float32)
                chk = (1.0 + kv * zs) > cs
                kz = plsc.all_reduce_population_count(chk).astype(jnp.float32)
                tausum = jnp.sum(jnp.where(chk, zs, 0.0))
                return (jnp.full((L,), tausum, jnp.float32) - 1.0) / kz

            def fallback_solve():
                @plsc.parallel_loop(0, nchunk, unroll=4)
                def _ex(i):
                    xz[pl.ds(i * L, L)] = jnp.exp(xz[pl.ds(i * L, L)])

                return bisect(xz, nchunk, lo0, zmax)

            tau = lax.cond(
                ncand <= L,
                sort_solve,
                lambda: lax.cond(
                    normal,
                    lambda: bisect(candz, ncc, lo0, zmax),
                    fallback_solve))

            def c_normal():
                def cf(i, _):
                    base = i * L
                    z = candz[pl.ds(base, L)]
                    pc = jnp.maximum(z - tau, 0.0)
                    mv = (base + iota) < ncand
                    plsc.store_scatter(
                        outb, [candi[pl.ds(base, L)]], pc, mask=mv)
                    return 0

                lax.fori_loop(0, ncc, cf, 0)

            def c_fallback():
                @plsc.parallel_loop(0, nchunk, unroll=8)
                def _cd(i):
                    z = xz[pl.ds(i * L, L)]
                    outb[pl.ds(i * L, L)] = jnp.maximum(z - tau, 0.0)

            lax.cond(normal, c_normal, c_fallback)
            out_copy(r).start()
            prev_normal = (normal, ncand, candi, r)

        out_copy(rpw - 1).wait()

    return k


def kernel(input):
    nrows, dim = input.shape
    return _sparsegen_rows(nrows, dim)(input)
